# baseline (speedup 1.0000x reference)
import functools

import jax
import jax.numpy as jnp
from jax import lax
from jax.experimental import pallas as pl
from jax.experimental.pallas import tpu as pltpu
from jax.experimental.pallas import tpu_sc as plsc

_N = 10000
_E = 320000
_D = 128
_O = 64
_G = 128

_NC = 2
_NS = 16
_NW = _NC * _NS
_CHUNK = 128
_RPW = 80
_E_PAD = _NW * _RPW * _CHUNK
_N_ACC = 10112
_RPS = _N_ACC // _NS
_HR = _RPW // 2

_BR = 1000
_NB = _N // _BR


def _make_agg():
    mesh = plsc.VectorSubcoreMesh(
        core_axis_name="c", subcore_axis_name="s",
        num_cores=_NC, num_subcores=_NS)

    @functools.partial(
        pl.kernel,
        out_type=jax.ShapeDtypeStruct((_NC, _N_ACC, _D), jnp.float32),
        mesh=mesh,
        scratch_types=[
            pltpu.VMEM_SHARED((_N_ACC, _D), jnp.float32),
            pltpu.VMEM((_HR, _CHUNK), jnp.int32),
            pltpu.VMEM((_HR, _CHUNK), jnp.int32),
            pltpu.VMEM((_CHUNK, _D), jnp.float32),
            pltpu.VMEM((_CHUNK, _D), jnp.float32),
            pltpu.SemaphoreType.DMA,
            pltpu.SemaphoreType.DMA,
        ],
    )
    def agg(h_hbm, srcp_hbm, dstp_hbm, zeros_hbm, out_hbm,
            accum, src_v, dst_v, rows_a, rows_b, sem_a, sem_b):
        c = lax.axis_index("c")
        s = lax.axis_index("s")
        wid = c * _NS + s
        pltpu.sync_copy(zeros_hbm.at[pl.ds(s * _RPS, _RPS)],
                        accum.at[pl.ds(s * _RPS, _RPS)])
        plsc.subcore_barrier()

        for h in range(2):
            pltpu.sync_copy(srcp_hbm.at[wid].at[pl.ds(h * _HR, _HR)], src_v)
            pltpu.sync_copy(dstp_hbm.at[wid].at[pl.ds(h * _HR, _HR)], dst_v)
            pltpu.async_copy(h_hbm.at[src_v.at[0]], rows_a, sem_a).wait()

            def body(g, carry):
                j = g * 2
                pltpu.async_copy(h_hbm.at[src_v.at[j + 1]], rows_b, sem_b)
                pltpu.sync_copy(rows_a, accum.at[dst_v.at[j]], add=True)
                pltpu.make_async_copy(h_hbm.at[src_v.at[j + 1]], rows_b,
                                      sem_b).wait()

                @pl.when(g + 1 < _HR // 2)
                def _():
                    pltpu.async_copy(h_hbm.at[src_v.at[j + 2]], rows_a, sem_a)

                pltpu.sync_copy(rows_b, accum.at[dst_v.at[j + 1]], add=True)

                @pl.when(g + 1 < _HR // 2)
                def _():
                    pltpu.make_async_copy(h_hbm.at[src_v.at[j + 2]], rows_a,
                                          sem_a).wait()

                return carry

            lax.fori_loop(0, _HR // 2, body, 0)
        plsc.subcore_barrier()
        pltpu.sync_copy(accum.at[pl.ds(s * _RPS, _RPS)],
                        out_hbm.at[c].at[pl.ds(s * _RPS, _RPS)])

    return agg


_agg = _make_agg()


def _mlp_body(eps_ref, h_ref, p_ref, w0_ref, b0_ref, w1_ref, b1_ref, out_ref):
    z = eps_ref[0, 0] * h_ref[...] + p_ref[0] + p_ref[1]
    t = jnp.dot(z, w0_ref[...], preferred_element_type=jnp.float32)
    t = jnp.maximum(t + b0_ref[...], 0.0)
    u = jnp.dot(t, w1_ref[...], preferred_element_type=jnp.float32)
    out_ref[...] = jnp.maximum(u + b1_ref[...], 0.0)


def _mlp(epsp, h, p, w0, b0, w1, b1):
    return pl.pallas_call(
        _mlp_body,
        grid=(_NB,),
        in_specs=[
            pl.BlockSpec((1, 1), lambda i: (0, 0), memory_space=pltpu.SMEM),
            pl.BlockSpec((_BR, _D), lambda i: (i, 0)),
            pl.BlockSpec((_NC, _BR, _D), lambda i: (0, i, 0)),
            pl.BlockSpec((_D, _D), lambda i: (0, 0)),
            pl.BlockSpec((1, _D), lambda i: (0, 0)),
            pl.BlockSpec((_D, _D), lambda i: (0, 0)),
            pl.BlockSpec((1, _D), lambda i: (0, 0)),
        ],
        out_specs=pl.BlockSpec((_BR, _D), lambda i: (i, 0)),
        out_shape=jax.ShapeDtypeStruct((_N, _D), jnp.float32),
    )(epsp, h, p, w0, b0, w1, b1)


def _pool_body(batch_ref, x_ref, h1_ref, h2_ref, wp0_ref, wp1_ref, wp2_ref,
               bsum_ref, out_ref):
    i = pl.program_id(0)
    s = jnp.dot(x_ref[...], wp0_ref[...], preferred_element_type=jnp.float32)
    s += jnp.dot(h1_ref[...], wp1_ref[...], preferred_element_type=jnp.float32)
    s += jnp.dot(h2_ref[...], wp2_ref[...], preferred_element_type=jnp.float32)
    bid = batch_ref[0, 0, :]
    onehot = (bid[:, None] == lax.broadcasted_iota(jnp.int32, (_BR, _G), 1))
    onehot = onehot.astype(jnp.float32)
    contrib = lax.dot_general(onehot, s, (((0,), (0,)), ((), ())),
                              preferred_element_type=jnp.float32)

    @pl.when(i == 0)
    def _():
        out_ref[...] = jnp.broadcast_to(bsum_ref[...], (_G, _O))

    out_ref[...] += contrib


def _pool(batch3, x, h1, h2, wp0, wp1, wp2, bsum):
    return pl.pallas_call(
        _pool_body,
        grid=(_NB,),
        in_specs=[
            pl.BlockSpec((1, 1, _BR), lambda i: (i, 0, 0)),
            pl.BlockSpec((_BR, _D), lambda i: (i, 0)),
            pl.BlockSpec((_BR, _D), lambda i: (i, 0)),
            pl.BlockSpec((_BR, _D), lambda i: (i, 0)),
            pl.BlockSpec((_D, _O), lambda i: (0, 0)),
            pl.BlockSpec((_D, _O), lambda i: (0, 0)),
            pl.BlockSpec((_D, _O), lambda i: (0, 0)),
            pl.BlockSpec((1, _O), lambda i: (0, 0)),
        ],
        out_specs=pl.BlockSpec((_G, _O), lambda i: (0, 0)),
        out_shape=jax.ShapeDtypeStruct((_G, _O), jnp.float32),
    )(batch3, x, h1, h2, wp0, wp1, wp2, bsum)


def kernel(x, edge_index, batch, params):
    src = edge_index[0]
    dst = edge_index[1]
    npad = _E_PAD - _E
    pad_src = jnp.zeros((npad,), jnp.int32)
    pad_dst = _N + (jnp.arange(npad, dtype=jnp.int32) % (_N_ACC - _N))
    srcp = jnp.concatenate([src, pad_src]).reshape(_NW, _RPW, _CHUNK)
    dstp = jnp.concatenate([dst, pad_dst]).reshape(_NW, _RPW, _CHUNK)
    zeros = jnp.zeros((_N_ACC, _D), jnp.float32)
    batch3 = batch.reshape(_NB, 1, _BR)

    cbn = 1.0 / jnp.sqrt(1.0 + 1e-5)
    folded = []
    for l in range(2):
        g0 = params[f"mlp_g{l}"] * cbn
        w0 = params[f"W0_{l}"] * g0[None, :]
        b0 = (params[f"b0_{l}"] * g0 + params[f"mlp_b{l}"]).reshape(1, _D)
        g1 = params[f"g{l}"] * cbn
        w1 = params[f"W1_{l}"] * g1[None, :]
        b1 = (params[f"b1_{l}"] * g1 + params[f"b{l}"]).reshape(1, _D)
        epsp = (1.0 + params[f"eps{l}"]).reshape(1, 1)
        folded.append((epsp, w0, b0, w1, b1))

    h = x
    hidden = [x]
    for l in range(2):
        p = _agg(h, srcp, dstp, zeros)
        epsp, w0, b0, w1, b1 = folded[l]
        h = _mlp(epsp, h, p, w0, b0, w1, b1)
        hidden.append(h)

    bsum = (params["bp0"] + params["bp1"] + params["bp2"]).reshape(1, _O)
    return _pool(batch3, hidden[0], hidden[1], hidden[2],
                 params["Wp0"], params["Wp1"], params["Wp2"], bsum)


# speedup vs baseline: 2.2900x; 2.2900x over previous
import functools

import jax
import jax.numpy as jnp
from jax import lax
from jax.experimental import pallas as pl
from jax.experimental.pallas import tpu as pltpu
from jax.experimental.pallas import tpu_sc as plsc

_N = 10000
_E = 320000
_D = 128
_O = 64
_G = 128

_NC = 2
_NS = 16
_NW = _NC * _NS
_CHUNK = 128
_PAIR_R = 160
_R0 = 104
_R1 = _PAIR_R - _R0
_IDX_ROWS = _NS * _PAIR_R
_IDX_PAD = 15 * _PAIR_R + 2 * _R0
_E_PAD = _IDX_ROWS * _CHUNK
_N_ACC = 10112
_RPS = _N_ACC // _NS

_BR = 1000
_NB = _N // _BR


def _make_agg():
    mesh = plsc.VectorSubcoreMesh(
        core_axis_name="c", subcore_axis_name="s",
        num_cores=_NC, num_subcores=_NS)

    @functools.partial(
        pl.kernel,
        out_type=jax.ShapeDtypeStruct((_NC, _N_ACC, _D), jnp.float32),
        mesh=mesh,
        scratch_types=[
            pltpu.VMEM_SHARED((_N_ACC, _D), jnp.float32),
            pltpu.VMEM((_R0, _CHUNK), jnp.int32),
            pltpu.VMEM((_R0, _CHUNK), jnp.int32),
            pltpu.VMEM((_CHUNK, _D), jnp.float32),
            pltpu.SemaphoreType.DMA,
        ],
    )
    def agg(h_hbm, srcp_hbm, dstp_hbm, zeros_hbm, out_hbm,
            accum, src_v, dst_v, rows_v, sem):
        c = lax.axis_index("c")
        s = lax.axis_index("s")
        pltpu.sync_copy(zeros_hbm.at[pl.ds(s * _RPS, _RPS)],
                        accum.at[pl.ds(s * _RPS, _RPS)])
        base = s * _PAIR_R + c * _R0
        trip = lax.select(c == 0, _R0, _R1)
        pltpu.sync_copy(srcp_hbm.at[pl.ds(base, _R0)], src_v)
        pltpu.sync_copy(dstp_hbm.at[pl.ds(base, _R0)], dst_v)
        plsc.subcore_barrier()

        def body(j, carry):
            pltpu.async_copy(h_hbm.at[src_v.at[j]], rows_v, sem).wait()
            pltpu.sync_copy(rows_v, accum.at[dst_v.at[j]], add=True)
            return carry

        lax.fori_loop(0, trip, body, 0)
        plsc.subcore_barrier()
        pltpu.sync_copy(accum.at[pl.ds(s * _RPS, _RPS)],
                        out_hbm.at[c].at[pl.ds(s * _RPS, _RPS)])

    return agg


_agg = _make_agg()


def _mlp_body(eps_ref, h_ref, p_ref, w0_ref, b0_ref, w1_ref, b1_ref, out_ref):
    z = eps_ref[0, 0] * h_ref[...] + p_ref[0] + p_ref[1]
    t = jnp.dot(z, w0_ref[...], preferred_element_type=jnp.float32)
    t = jnp.maximum(t + b0_ref[...], 0.0)
    u = jnp.dot(t, w1_ref[...], preferred_element_type=jnp.float32)
    out_ref[...] = jnp.maximum(u + b1_ref[...], 0.0)


def _mlp(epsp, h, p, w0, b0, w1, b1):
    return pl.pallas_call(
        _mlp_body,
        grid=(_NB,),
        in_specs=[
            pl.BlockSpec((1, 1), lambda i: (0, 0), memory_space=pltpu.SMEM),
            pl.BlockSpec((_BR, _D), lambda i: (i, 0)),
            pl.BlockSpec((_NC, _BR, _D), lambda i: (0, i, 0)),
            pl.BlockSpec((_D, _D), lambda i: (0, 0)),
            pl.BlockSpec((1, _D), lambda i: (0, 0)),
            pl.BlockSpec((_D, _D), lambda i: (0, 0)),
            pl.BlockSpec((1, _D), lambda i: (0, 0)),
        ],
        out_specs=pl.BlockSpec((_BR, _D), lambda i: (i, 0)),
        out_shape=jax.ShapeDtypeStruct((_N, _D), jnp.float32),
    )(epsp, h, p, w0, b0, w1, b1)


def _pool_body(batch_ref, x_ref, h1_ref, h2_ref, wp0_ref, wp1_ref, wp2_ref,
               bsum_ref, out_ref):
    i = pl.program_id(0)
    s = jnp.dot(x_ref[...], wp0_ref[...], preferred_element_type=jnp.float32)
    s += jnp.dot(h1_ref[...], wp1_ref[...], preferred_element_type=jnp.float32)
    s += jnp.dot(h2_ref[...], wp2_ref[...], preferred_element_type=jnp.float32)
    bid = batch_ref[0, 0, :]
    onehot = (bid[:, None] == lax.broadcasted_iota(jnp.int32, (_BR, _G), 1))
    onehot = onehot.astype(jnp.float32)
    contrib = lax.dot_general(onehot, s, (((0,), (0,)), ((), ())),
                              preferred_element_type=jnp.float32)

    @pl.when(i == 0)
    def _():
        out_ref[...] = jnp.broadcast_to(bsum_ref[...], (_G, _O))

    out_ref[...] += contrib


def _pool(batch3, x, h1, h2, wp0, wp1, wp2, bsum):
    return pl.pallas_call(
        _pool_body,
        grid=(_NB,),
        in_specs=[
            pl.BlockSpec((1, 1, _BR), lambda i: (i, 0, 0)),
            pl.BlockSpec((_BR, _D), lambda i: (i, 0)),
            pl.BlockSpec((_BR, _D), lambda i: (i, 0)),
            pl.BlockSpec((_BR, _D), lambda i: (i, 0)),
            pl.BlockSpec((_D, _O), lambda i: (0, 0)),
            pl.BlockSpec((_D, _O), lambda i: (0, 0)),
            pl.BlockSpec((_D, _O), lambda i: (0, 0)),
            pl.BlockSpec((1, _O), lambda i: (0, 0)),
        ],
        out_specs=pl.BlockSpec((_G, _O), lambda i: (0, 0)),
        out_shape=jax.ShapeDtypeStruct((_G, _O), jnp.float32),
    )(batch3, x, h1, h2, wp0, wp1, wp2, bsum)


def kernel(x, edge_index, batch, params):
    src = edge_index[0]
    dst = edge_index[1]
    npad = _E_PAD - _E
    pad_src = jnp.arange(npad, dtype=jnp.int32) % _N
    pad_dst = _N + (jnp.arange(npad, dtype=jnp.int32) % (_N_ACC - _N))
    srcp = jnp.concatenate([src, pad_src]).reshape(_IDX_ROWS, _CHUNK)
    dstp = jnp.concatenate([dst, pad_dst]).reshape(_IDX_ROWS, _CHUNK)
    tail = _IDX_PAD - _IDX_ROWS
    srcp = jnp.concatenate([srcp, jnp.zeros((tail, _CHUNK), jnp.int32)])
    dstp = jnp.concatenate([dstp, jnp.full((tail, _CHUNK), _N, jnp.int32)])
    zeros = jnp.zeros((_N_ACC, _D), jnp.float32)
    batch3 = batch.reshape(_NB, 1, _BR)

    cbn = 1.0 / jnp.sqrt(1.0 + 1e-5)
    folded = []
    for l in range(2):
        g0 = params[f"mlp_g{l}"] * cbn
        w0 = params[f"W0_{l}"] * g0[None, :]
        b0 = (params[f"b0_{l}"] * g0 + params[f"mlp_b{l}"]).reshape(1, _D)
        g1 = params[f"g{l}"] * cbn
        w1 = params[f"W1_{l}"] * g1[None, :]
        b1 = (params[f"b1_{l}"] * g1 + params[f"b{l}"]).reshape(1, _D)
        epsp = (1.0 + params[f"eps{l}"]).reshape(1, 1)
        folded.append((epsp, w0, b0, w1, b1))

    h = x
    hidden = [x]
    for l in range(2):
        p = _agg(h, srcp, dstp, zeros)
        epsp, w0, b0, w1, b1 = folded[l]
        h = _mlp(epsp, h, p, w0, b0, w1, b1)
        hidden.append(h)

    bsum = (params["bp0"] + params["bp1"] + params["bp2"]).reshape(1, _O)
    return _pool(batch3, hidden[0], hidden[1], hidden[2],
                 params["Wp0"], params["Wp1"], params["Wp2"], bsum)


# speedup vs baseline: 2.6547x; 1.1592x over previous
import functools

import jax
import jax.numpy as jnp
from jax import lax
from jax.experimental import pallas as pl
from jax.experimental.pallas import tpu as pltpu
from jax.experimental.pallas import tpu_sc as plsc

_N = 10000
_E = 320000
_D = 128
_O = 64
_G = 128

_NC = 2
_NS = 16
_NW = _NC * _NS
_CHUNK = 128
_PAIR_R = 160
_R0 = 104
_R1 = _PAIR_R - _R0
_IDX_ROWS = _NS * _PAIR_R
_IDX_PAD = 15 * _PAIR_R + 2 * _R0
_E_PAD = _IDX_ROWS * _CHUNK
_HB = 56
_N_ACC = 10112
_RPS = _N_ACC // _NS

_BR = 1000
_NB = _N // _BR


def _make_agg():
    mesh = plsc.VectorSubcoreMesh(
        core_axis_name="c", subcore_axis_name="s",
        num_cores=_NC, num_subcores=_NS)

    @functools.partial(
        pl.kernel,
        out_type=jax.ShapeDtypeStruct((_NC, _N_ACC, _D), jnp.float32),
        mesh=mesh,
        scratch_types=[
            pltpu.VMEM_SHARED((_N_ACC, _D), jnp.float32),
            pltpu.VMEM((_HB, _CHUNK), jnp.int32),
            pltpu.VMEM((_HB, _CHUNK), jnp.int32),
            pltpu.VMEM((_CHUNK, _D), jnp.float32),
            pltpu.VMEM((_CHUNK, _D), jnp.float32),
            pltpu.SemaphoreType.DMA,
            pltpu.SemaphoreType.DMA,
            pltpu.SemaphoreType.DMA,
            pltpu.SemaphoreType.DMA,
        ],
    )
    def agg(h_hbm, srcp_hbm, dstp_hbm, zeros_hbm, out_hbm,
            accum, src_v, dst_v, rows_a, rows_b,
            sem_ga, sem_gb, sem_sa, sem_sb):
        c = lax.axis_index("c")
        s = lax.axis_index("s")
        pltpu.sync_copy(zeros_hbm.at[pl.ds(s * _RPS, _RPS)],
                        accum.at[pl.ds(s * _RPS, _RPS)])
        base = s * _PAIR_R + c * _R0
        plsc.subcore_barrier()

        for hh in range(2):
            off = base + hh * _HB * (1 - c)
            pltpu.sync_copy(srcp_hbm.at[pl.ds(off, _HB)], src_v)
            pltpu.sync_copy(dstp_hbm.at[pl.ds(off, _HB)], dst_v)
            if hh == 0:
                trip = _HB // 2
            else:
                trip = lax.select(c == 0, (_R0 - _HB) // 2, 0)

            def body(g, carry):
                j = g * 2
                ga = pltpu.async_copy(h_hbm.at[src_v.at[j]], rows_a, sem_ga)
                gb = pltpu.async_copy(h_hbm.at[src_v.at[j + 1]], rows_b,
                                      sem_gb)
                ga.wait()
                sa = pltpu.async_copy(rows_a, accum.at[dst_v.at[j]], sem_sa,
                                      add=True)
                gb.wait()
                sb = pltpu.async_copy(rows_b, accum.at[dst_v.at[j + 1]],
                                      sem_sb, add=True)
                sa.wait()
                sb.wait()
                return carry

            lax.fori_loop(0, trip, body, 0)
        plsc.subcore_barrier()
        pltpu.sync_copy(accum.at[pl.ds(s * _RPS, _RPS)],
                        out_hbm.at[c].at[pl.ds(s * _RPS, _RPS)])

    return agg


_agg = _make_agg()


def _mlp_body(eps_ref, h_ref, p_ref, w0_ref, b0_ref, w1_ref, b1_ref, out_ref):
    z = eps_ref[0, 0] * h_ref[...] + p_ref[0] + p_ref[1]
    t = jnp.dot(z, w0_ref[...], preferred_element_type=jnp.float32)
    t = jnp.maximum(t + b0_ref[...], 0.0)
    u = jnp.dot(t, w1_ref[...], preferred_element_type=jnp.float32)
    out_ref[...] = jnp.maximum(u + b1_ref[...], 0.0)


def _mlp(epsp, h, p, w0, b0, w1, b1):
    return pl.pallas_call(
        _mlp_body,
        grid=(_NB,),
        in_specs=[
            pl.BlockSpec((1, 1), lambda i: (0, 0), memory_space=pltpu.SMEM),
            pl.BlockSpec((_BR, _D), lambda i: (i, 0)),
            pl.BlockSpec((_NC, _BR, _D), lambda i: (0, i, 0)),
            pl.BlockSpec((_D, _D), lambda i: (0, 0)),
            pl.BlockSpec((1, _D), lambda i: (0, 0)),
            pl.BlockSpec((_D, _D), lambda i: (0, 0)),
            pl.BlockSpec((1, _D), lambda i: (0, 0)),
        ],
        out_specs=pl.BlockSpec((_BR, _D), lambda i: (i, 0)),
        out_shape=jax.ShapeDtypeStruct((_N, _D), jnp.float32),
    )(epsp, h, p, w0, b0, w1, b1)


def _pool_body(batch_ref, x_ref, h1_ref, h2_ref, wp0_ref, wp1_ref, wp2_ref,
               bsum_ref, out_ref):
    i = pl.program_id(0)
    s = jnp.dot(x_ref[...], wp0_ref[...], preferred_element_type=jnp.float32)
    s += jnp.dot(h1_ref[...], wp1_ref[...], preferred_element_type=jnp.float32)
    s += jnp.dot(h2_ref[...], wp2_ref[...], preferred_element_type=jnp.float32)
    bid = batch_ref[0, 0, :]
    onehot = (bid[:, None] == lax.broadcasted_iota(jnp.int32, (_BR, _G), 1))
    onehot = onehot.astype(jnp.float32)
    contrib = lax.dot_general(onehot, s, (((0,), (0,)), ((), ())),
                              preferred_element_type=jnp.float32)

    @pl.when(i == 0)
    def _():
        out_ref[...] = jnp.broadcast_to(bsum_ref[...], (_G, _O))

    out_ref[...] += contrib


def _pool(batch3, x, h1, h2, wp0, wp1, wp2, bsum):
    return pl.pallas_call(
        _pool_body,
        grid=(_NB,),
        in_specs=[
            pl.BlockSpec((1, 1, _BR), lambda i: (i, 0, 0)),
            pl.BlockSpec((_BR, _D), lambda i: (i, 0)),
            pl.BlockSpec((_BR, _D), lambda i: (i, 0)),
            pl.BlockSpec((_BR, _D), lambda i: (i, 0)),
            pl.BlockSpec((_D, _O), lambda i: (0, 0)),
            pl.BlockSpec((_D, _O), lambda i: (0, 0)),
            pl.BlockSpec((_D, _O), lambda i: (0, 0)),
            pl.BlockSpec((1, _O), lambda i: (0, 0)),
        ],
        out_specs=pl.BlockSpec((_G, _O), lambda i: (0, 0)),
        out_shape=jax.ShapeDtypeStruct((_G, _O), jnp.float32),
    )(batch3, x, h1, h2, wp0, wp1, wp2, bsum)


def kernel(x, edge_index, batch, params):
    src = edge_index[0]
    dst = edge_index[1]
    npad = _E_PAD - _E
    pad_src = jnp.arange(npad, dtype=jnp.int32) % _N
    pad_dst = _N + (jnp.arange(npad, dtype=jnp.int32) % (_N_ACC - _N))
    srcp = jnp.concatenate([src, pad_src]).reshape(_IDX_ROWS, _CHUNK)
    dstp = jnp.concatenate([dst, pad_dst]).reshape(_IDX_ROWS, _CHUNK)
    tail = _IDX_PAD - _IDX_ROWS
    srcp = jnp.concatenate([srcp, jnp.zeros((tail, _CHUNK), jnp.int32)])
    dstp = jnp.concatenate([dstp, jnp.full((tail, _CHUNK), _N, jnp.int32)])
    zeros = jnp.zeros((_N_ACC, _D), jnp.float32)
    batch3 = batch.reshape(_NB, 1, _BR)

    cbn = 1.0 / jnp.sqrt(1.0 + 1e-5)
    folded = []
    for l in range(2):
        g0 = params[f"mlp_g{l}"] * cbn
        w0 = params[f"W0_{l}"] * g0[None, :]
        b0 = (params[f"b0_{l}"] * g0 + params[f"mlp_b{l}"]).reshape(1, _D)
        g1 = params[f"g{l}"] * cbn
        w1 = params[f"W1_{l}"] * g1[None, :]
        b1 = (params[f"b1_{l}"] * g1 + params[f"b{l}"]).reshape(1, _D)
        epsp = (1.0 + params[f"eps{l}"]).reshape(1, 1)
        folded.append((epsp, w0, b0, w1, b1))

    h = x
    hidden = [x]
    for l in range(2):
        p = _agg(h, srcp, dstp, zeros)
        epsp, w0, b0, w1, b1 = folded[l]
        h = _mlp(epsp, h, p, w0, b0, w1, b1)
        hidden.append(h)

    bsum = (params["bp0"] + params["bp1"] + params["bp2"]).reshape(1, _O)
    return _pool(batch3, hidden[0], hidden[1], hidden[2],
                 params["Wp0"], params["Wp1"], params["Wp2"], bsum)


# speedup vs baseline: 2.9882x; 1.1256x over previous
import functools

import jax
import jax.numpy as jnp
from jax import lax
from jax.experimental import pallas as pl
from jax.experimental.pallas import tpu as pltpu
from jax.experimental.pallas import tpu_sc as plsc

_N = 10000
_E = 320000
_D = 128
_O = 64
_G = 128

_NC = 2
_NS = 16
_NW = _NC * _NS
_CHUNK = 128
_PAIR_R = 160
_R0 = 88
_R1 = _PAIR_R - _R0
_IDX_ROWS = _NS * _PAIR_R
_HB = 48
_IDX_PAD = 15 * _PAIR_R + _R0 + 2 * _HB
_E_PAD = _IDX_ROWS * _CHUNK
_N_ACC = 10112
_RPS = _N_ACC // _NS

_BR = 1000
_NB = _N // _BR


def _make_agg():
    mesh = plsc.VectorSubcoreMesh(
        core_axis_name="c", subcore_axis_name="s",
        num_cores=_NC, num_subcores=_NS)

    @functools.partial(
        pl.kernel,
        out_type=jax.ShapeDtypeStruct((_NC, _N_ACC, _D), jnp.float32),
        mesh=mesh,
        scratch_types=[
            pltpu.VMEM_SHARED((_N_ACC, _D), jnp.float32),
            pltpu.VMEM((_HB, _CHUNK), jnp.int32),
            pltpu.VMEM((_HB, _CHUNK), jnp.int32),
            pltpu.VMEM((_CHUNK, _D), jnp.float32),
            pltpu.VMEM((_CHUNK, _D), jnp.float32),
            pltpu.SemaphoreType.DMA,
            pltpu.SemaphoreType.DMA,
            pltpu.SemaphoreType.DMA,
            pltpu.SemaphoreType.DMA,
        ],
    )
    def agg(h_hbm, srcp_hbm, dstp_hbm, zeros_hbm, out_hbm,
            accum, src_v, dst_v, rows_a, rows_b,
            sem_ga, sem_gb, sem_sa, sem_sb):
        c = lax.axis_index("c")
        s = lax.axis_index("s")
        pltpu.sync_copy(zeros_hbm.at[pl.ds(s * _RPS, _RPS)],
                        accum.at[pl.ds(s * _RPS, _RPS)])
        base = s * _PAIR_R + c * _R0
        plsc.subcore_barrier()

        for hh in range(2):
            off = base + hh * _HB
            pltpu.sync_copy(srcp_hbm.at[pl.ds(off, _HB)], src_v)
            pltpu.sync_copy(dstp_hbm.at[pl.ds(off, _HB)], dst_v)
            if hh == 0:
                trip = _HB // 2
            else:
                trip = lax.select(c == 0, (_R0 - _HB) // 2, (_R1 - _HB) // 2)

            def body(g, carry):
                j = g * 2
                ga = pltpu.async_copy(h_hbm.at[src_v.at[j]], rows_a, sem_ga)
                gb = pltpu.async_copy(h_hbm.at[src_v.at[j + 1]], rows_b,
                                      sem_gb)
                ga.wait()
                sa = pltpu.async_copy(rows_a, accum.at[dst_v.at[j]], sem_sa,
                                      add=True)
                gb.wait()
                sb = pltpu.async_copy(rows_b, accum.at[dst_v.at[j + 1]],
                                      sem_sb, add=True)
                sa.wait()
                sb.wait()
                return carry

            lax.fori_loop(0, trip, body, 0)
        plsc.subcore_barrier()
        pltpu.sync_copy(accum.at[pl.ds(s * _RPS, _RPS)],
                        out_hbm.at[c].at[pl.ds(s * _RPS, _RPS)])

    return agg


_agg = _make_agg()


def _mlp_body(eps_ref, h_ref, p_ref, w0_ref, b0_ref, w1_ref, b1_ref, out_ref):
    z = eps_ref[0, 0] * h_ref[...] + p_ref[0] + p_ref[1]
    t = jnp.dot(z, w0_ref[...], preferred_element_type=jnp.float32)
    t = jnp.maximum(t + b0_ref[...], 0.0)
    u = jnp.dot(t, w1_ref[...], preferred_element_type=jnp.float32)
    out_ref[...] = jnp.maximum(u + b1_ref[...], 0.0)


def _mlp(epsp, h, p, w0, b0, w1, b1):
    return pl.pallas_call(
        _mlp_body,
        grid=(_NB,),
        in_specs=[
            pl.BlockSpec((1, 1), lambda i: (0, 0), memory_space=pltpu.SMEM),
            pl.BlockSpec((_BR, _D), lambda i: (i, 0)),
            pl.BlockSpec((_NC, _BR, _D), lambda i: (0, i, 0)),
            pl.BlockSpec((_D, _D), lambda i: (0, 0)),
            pl.BlockSpec((1, _D), lambda i: (0, 0)),
            pl.BlockSpec((_D, _D), lambda i: (0, 0)),
            pl.BlockSpec((1, _D), lambda i: (0, 0)),
        ],
        out_specs=pl.BlockSpec((_BR, _D), lambda i: (i, 0)),
        out_shape=jax.ShapeDtypeStruct((_N, _D), jnp.float32),
    )(epsp, h, p, w0, b0, w1, b1)


def _pool_body(batch_ref, x_ref, h1_ref, h2_ref, wp0_ref, wp1_ref, wp2_ref,
               bsum_ref, out_ref):
    i = pl.program_id(0)
    s = jnp.dot(x_ref[...], wp0_ref[...], preferred_element_type=jnp.float32)
    s += jnp.dot(h1_ref[...], wp1_ref[...], preferred_element_type=jnp.float32)
    s += jnp.dot(h2_ref[...], wp2_ref[...], preferred_element_type=jnp.float32)
    bid = batch_ref[0, 0, :]
    onehot = (bid[:, None] == lax.broadcasted_iota(jnp.int32, (_BR, _G), 1))
    onehot = onehot.astype(jnp.float32)
    contrib = lax.dot_general(onehot, s, (((0,), (0,)), ((), ())),
                              preferred_element_type=jnp.float32)

    @pl.when(i == 0)
    def _():
        out_ref[...] = jnp.broadcast_to(bsum_ref[...], (_G, _O))

    out_ref[...] += contrib


def _pool(batch3, x, h1, h2, wp0, wp1, wp2, bsum):
    return pl.pallas_call(
        _pool_body,
        grid=(_NB,),
        in_specs=[
            pl.BlockSpec((1, 1, _BR), lambda i: (i, 0, 0)),
            pl.BlockSpec((_BR, _D), lambda i: (i, 0)),
            pl.BlockSpec((_BR, _D), lambda i: (i, 0)),
            pl.BlockSpec((_BR, _D), lambda i: (i, 0)),
            pl.BlockSpec((_D, _O), lambda i: (0, 0)),
            pl.BlockSpec((_D, _O), lambda i: (0, 0)),
            pl.BlockSpec((_D, _O), lambda i: (0, 0)),
            pl.BlockSpec((1, _O), lambda i: (0, 0)),
        ],
        out_specs=pl.BlockSpec((_G, _O), lambda i: (0, 0)),
        out_shape=jax.ShapeDtypeStruct((_G, _O), jnp.float32),
    )(batch3, x, h1, h2, wp0, wp1, wp2, bsum)


def kernel(x, edge_index, batch, params):
    src = edge_index[0]
    dst = edge_index[1]
    npad = _E_PAD - _E
    pad_src = jnp.arange(npad, dtype=jnp.int32) % _N
    pad_dst = _N + (jnp.arange(npad, dtype=jnp.int32) % (_N_ACC - _N))
    srcp = jnp.concatenate([src, pad_src]).reshape(_IDX_ROWS, _CHUNK)
    dstp = jnp.concatenate([dst, pad_dst]).reshape(_IDX_ROWS, _CHUNK)
    tail = _IDX_PAD - _IDX_ROWS
    srcp = jnp.concatenate([srcp, jnp.zeros((tail, _CHUNK), jnp.int32)])
    dstp = jnp.concatenate([dstp, jnp.full((tail, _CHUNK), _N, jnp.int32)])
    zeros = jnp.zeros((_N_ACC, _D), jnp.float32)
    batch3 = batch.reshape(_NB, 1, _BR)

    cbn = 1.0 / jnp.sqrt(1.0 + 1e-5)
    folded = []
    for l in range(2):
        g0 = params[f"mlp_g{l}"] * cbn
        w0 = params[f"W0_{l}"] * g0[None, :]
        b0 = (params[f"b0_{l}"] * g0 + params[f"mlp_b{l}"]).reshape(1, _D)
        g1 = params[f"g{l}"] * cbn
        w1 = params[f"W1_{l}"] * g1[None, :]
        b1 = (params[f"b1_{l}"] * g1 + params[f"b{l}"]).reshape(1, _D)
        epsp = (1.0 + params[f"eps{l}"]).reshape(1, 1)
        folded.append((epsp, w0, b0, w1, b1))

    h = x
    hidden = [x]
    for l in range(2):
        p = _agg(h, srcp, dstp, zeros)
        epsp, w0, b0, w1, b1 = folded[l]
        h = _mlp(epsp, h, p, w0, b0, w1, b1)
        hidden.append(h)

    bsum = (params["bp0"] + params["bp1"] + params["bp2"]).reshape(1, _O)
    return _pool(batch3, hidden[0], hidden[1], hidden[2],
                 params["Wp0"], params["Wp1"], params["Wp2"], bsum)


# speedup vs baseline: 3.2904x; 1.1011x over previous
import functools

import jax
import jax.numpy as jnp
from jax import lax
from jax.experimental import pallas as pl
from jax.experimental.pallas import tpu as pltpu
from jax.experimental.pallas import tpu_sc as plsc

_N = 10000
_E = 320000
_D = 128
_O = 64
_G = 128

_NC = 2
_NS = 16
_NW = _NC * _NS
_CHUNK = 128
_PAIR_R = 160
_R0 = 88
_R1 = _PAIR_R - _R0
_IDX_ROWS = _NS * _PAIR_R
_HB = 48
_HBL = _HB + 8
_IDX_PAD = 15 * _PAIR_R + _R0 + _HB + _HBL
_E_PAD = _IDX_ROWS * _CHUNK
_N_ACC = 10112
_RPS = _N_ACC // _NS

_BR = 1000
_NB = _N // _BR


def _make_agg():
    mesh = plsc.VectorSubcoreMesh(
        core_axis_name="c", subcore_axis_name="s",
        num_cores=_NC, num_subcores=_NS)

    @functools.partial(
        pl.kernel,
        out_type=jax.ShapeDtypeStruct((_NC, _N_ACC, _D), jnp.float32),
        mesh=mesh,
        scratch_types=[
            pltpu.VMEM_SHARED((_N_ACC, _D), jnp.float32),
            pltpu.VMEM((_HBL, _CHUNK), jnp.int32),
            pltpu.VMEM((_HBL, _CHUNK), jnp.int32),
            pltpu.VMEM((_CHUNK, _D), jnp.float32),
            pltpu.VMEM((_CHUNK, _D), jnp.float32),
            pltpu.SemaphoreType.DMA,
            pltpu.SemaphoreType.DMA,
            pltpu.SemaphoreType.DMA,
            pltpu.SemaphoreType.DMA,
        ],
    )
    def agg(h_hbm, srcp_hbm, dstp_hbm, zeros_hbm, out_hbm,
            accum, src_v, dst_v, rows_a, rows_b,
            sem_ga, sem_gb, sem_sa, sem_sb):
        c = lax.axis_index("c")
        s = lax.axis_index("s")
        pltpu.sync_copy(zeros_hbm.at[pl.ds(s * _RPS, _RPS)],
                        accum.at[pl.ds(s * _RPS, _RPS)])
        base = s * _PAIR_R + c * _R0
        plsc.subcore_barrier()

        for hh in range(2):
            off = base + hh * _HB
            pltpu.sync_copy(srcp_hbm.at[pl.ds(off, _HBL)], src_v)
            pltpu.sync_copy(dstp_hbm.at[pl.ds(off, _HBL)], dst_v)
            if hh == 0:
                trip = _HB // 2
            else:
                trip = lax.select(c == 0, (_R0 - _HB) // 2, (_R1 - _HB) // 2)

            pltpu.async_copy(h_hbm.at[src_v.at[0]], rows_a, sem_ga)

            def body(g, carry):
                j = g * 2
                pltpu.make_async_copy(h_hbm.at[src_v.at[j]], rows_a,
                                      sem_ga).wait()
                gb = pltpu.async_copy(h_hbm.at[src_v.at[j + 1]], rows_b,
                                      sem_gb)
                sa = pltpu.async_copy(rows_a, accum.at[dst_v.at[j]], sem_sa,
                                      add=True)
                gb.wait()
                sb = pltpu.async_copy(rows_b, accum.at[dst_v.at[j + 1]],
                                      sem_sb, add=True)
                sa.wait()
                pltpu.async_copy(h_hbm.at[src_v.at[j + 2]], rows_a, sem_ga)
                sb.wait()
                return carry

            lax.fori_loop(0, trip, body, 0)
            pltpu.make_async_copy(h_hbm.at[src_v.at[0]], rows_a, sem_ga).wait()
        plsc.subcore_barrier()
        pltpu.sync_copy(accum.at[pl.ds(s * _RPS, _RPS)],
                        out_hbm.at[c].at[pl.ds(s * _RPS, _RPS)])

    return agg


_agg = _make_agg()


def _mlp_body(eps_ref, h_ref, p_ref, w0_ref, b0_ref, w1_ref, b1_ref, out_ref):
    z = eps_ref[0, 0] * h_ref[...] + p_ref[0] + p_ref[1]
    t = jnp.dot(z, w0_ref[...], preferred_element_type=jnp.float32)
    t = jnp.maximum(t + b0_ref[...], 0.0)
    u = jnp.dot(t, w1_ref[...], preferred_element_type=jnp.float32)
    out_ref[...] = jnp.maximum(u + b1_ref[...], 0.0)


def _mlp(epsp, h, p, w0, b0, w1, b1):
    return pl.pallas_call(
        _mlp_body,
        grid=(_NB,),
        in_specs=[
            pl.BlockSpec((1, 1), lambda i: (0, 0), memory_space=pltpu.SMEM),
            pl.BlockSpec((_BR, _D), lambda i: (i, 0)),
            pl.BlockSpec((_NC, _BR, _D), lambda i: (0, i, 0)),
            pl.BlockSpec((_D, _D), lambda i: (0, 0)),
            pl.BlockSpec((1, _D), lambda i: (0, 0)),
            pl.BlockSpec((_D, _D), lambda i: (0, 0)),
            pl.BlockSpec((1, _D), lambda i: (0, 0)),
        ],
        out_specs=pl.BlockSpec((_BR, _D), lambda i: (i, 0)),
        out_shape=jax.ShapeDtypeStruct((_N, _D), jnp.float32),
    )(epsp, h, p, w0, b0, w1, b1)


def _pool_body(batch_ref, x_ref, h1_ref, h2_ref, wp0_ref, wp1_ref, wp2_ref,
               bsum_ref, out_ref):
    i = pl.program_id(0)
    s = jnp.dot(x_ref[...], wp0_ref[...], preferred_element_type=jnp.float32)
    s += jnp.dot(h1_ref[...], wp1_ref[...], preferred_element_type=jnp.float32)
    s += jnp.dot(h2_ref[...], wp2_ref[...], preferred_element_type=jnp.float32)
    bid = batch_ref[0, 0, :]
    onehot = (bid[:, None] == lax.broadcasted_iota(jnp.int32, (_BR, _G), 1))
    onehot = onehot.astype(jnp.float32)
    contrib = lax.dot_general(onehot, s, (((0,), (0,)), ((), ())),
                              preferred_element_type=jnp.float32)

    @pl.when(i == 0)
    def _():
        out_ref[...] = jnp.broadcast_to(bsum_ref[...], (_G, _O))

    out_ref[...] += contrib


def _pool(batch3, x, h1, h2, wp0, wp1, wp2, bsum):
    return pl.pallas_call(
        _pool_body,
        grid=(_NB,),
        in_specs=[
            pl.BlockSpec((1, 1, _BR), lambda i: (i, 0, 0)),
            pl.BlockSpec((_BR, _D), lambda i: (i, 0)),
            pl.BlockSpec((_BR, _D), lambda i: (i, 0)),
            pl.BlockSpec((_BR, _D), lambda i: (i, 0)),
            pl.BlockSpec((_D, _O), lambda i: (0, 0)),
            pl.BlockSpec((_D, _O), lambda i: (0, 0)),
            pl.BlockSpec((_D, _O), lambda i: (0, 0)),
            pl.BlockSpec((1, _O), lambda i: (0, 0)),
        ],
        out_specs=pl.BlockSpec((_G, _O), lambda i: (0, 0)),
        out_shape=jax.ShapeDtypeStruct((_G, _O), jnp.float32),
    )(batch3, x, h1, h2, wp0, wp1, wp2, bsum)


def kernel(x, edge_index, batch, params):
    src = edge_index[0]
    dst = edge_index[1]
    npad = _E_PAD - _E
    pad_src = jnp.arange(npad, dtype=jnp.int32) % _N
    pad_dst = _N + (jnp.arange(npad, dtype=jnp.int32) % (_N_ACC - _N))
    srcp = jnp.concatenate([src, pad_src]).reshape(_IDX_ROWS, _CHUNK)
    dstp = jnp.concatenate([dst, pad_dst]).reshape(_IDX_ROWS, _CHUNK)
    tail = _IDX_PAD - _IDX_ROWS
    srcp = jnp.concatenate([srcp, jnp.zeros((tail, _CHUNK), jnp.int32)])
    dstp = jnp.concatenate([dstp, jnp.full((tail, _CHUNK), _N, jnp.int32)])
    zeros = jnp.zeros((_N_ACC, _D), jnp.float32)
    batch3 = batch.reshape(_NB, 1, _BR)

    cbn = 1.0 / jnp.sqrt(1.0 + 1e-5)
    folded = []
    for l in range(2):
        g0 = params[f"mlp_g{l}"] * cbn
        w0 = params[f"W0_{l}"] * g0[None, :]
        b0 = (params[f"b0_{l}"] * g0 + params[f"mlp_b{l}"]).reshape(1, _D)
        g1 = params[f"g{l}"] * cbn
        w1 = params[f"W1_{l}"] * g1[None, :]
        b1 = (params[f"b1_{l}"] * g1 + params[f"b{l}"]).reshape(1, _D)
        epsp = (1.0 + params[f"eps{l}"]).reshape(1, 1)
        folded.append((epsp, w0, b0, w1, b1))

    h = x
    hidden = [x]
    for l in range(2):
        p = _agg(h, srcp, dstp, zeros)
        epsp, w0, b0, w1, b1 = folded[l]
        h = _mlp(epsp, h, p, w0, b0, w1, b1)
        hidden.append(h)

    bsum = (params["bp0"] + params["bp1"] + params["bp2"]).reshape(1, _O)
    return _pool(batch3, hidden[0], hidden[1], hidden[2],
                 params["Wp0"], params["Wp1"], params["Wp2"], bsum)


# speedup vs baseline: 3.4809x; 1.0579x over previous
import functools

import jax
import jax.numpy as jnp
from jax import lax
from jax.experimental import pallas as pl
from jax.experimental.pallas import tpu as pltpu
from jax.experimental.pallas import tpu_sc as plsc

_N = 10000
_E = 320000
_D = 128
_O = 64
_G = 128

_NC = 2
_NS = 16
_NW = _NC * _NS
_CHUNK = 64
_PAIR_R = 320
_R0 = 160
_R1 = _PAIR_R - _R0
_IDX_ROWS = _NS * _PAIR_R
_HB = 64
_IDX_PAD = 15 * _PAIR_R + _R0 + 3 * _HB
_E_PAD = _IDX_ROWS * _CHUNK
_N_ACC = 10112
_RPS = _N_ACC // _NS

_BR = 1000
_NB = _N // _BR


def _make_agg():
    mesh = plsc.VectorSubcoreMesh(
        core_axis_name="c", subcore_axis_name="s",
        num_cores=_NC, num_subcores=_NS)

    @functools.partial(
        pl.kernel,
        out_type=jax.ShapeDtypeStruct((_NC, _N_ACC, _D), jnp.float32),
        mesh=mesh,
        scratch_types=[
            pltpu.VMEM_SHARED((_N_ACC, _D), jnp.float32),
            pltpu.VMEM((_HB, _CHUNK), jnp.int32),
            pltpu.VMEM((_HB, _CHUNK), jnp.int32),
            pltpu.VMEM((_CHUNK, _D), jnp.float32),
            pltpu.VMEM((_CHUNK, _D), jnp.float32),
            pltpu.VMEM((_CHUNK, _D), jnp.float32),
            pltpu.VMEM((_CHUNK, _D), jnp.float32),
            pltpu.SemaphoreType.DMA,
            pltpu.SemaphoreType.DMA,
            pltpu.SemaphoreType.DMA,
            pltpu.SemaphoreType.DMA,
            pltpu.SemaphoreType.DMA,
            pltpu.SemaphoreType.DMA,
            pltpu.SemaphoreType.DMA,
            pltpu.SemaphoreType.DMA,
        ],
    )
    def agg(h_hbm, srcp_hbm, dstp_hbm, zeros_hbm, out_hbm,
            accum, src_v, dst_v, rows_a, rows_b, rows_c, rows_d,
            sem_ga, sem_gb, sem_gc, sem_gd,
            sem_sa, sem_sb, sem_sc, sem_sd):
        c = lax.axis_index("c")
        s = lax.axis_index("s")
        pltpu.sync_copy(zeros_hbm.at[pl.ds(s * _RPS, _RPS)],
                        accum.at[pl.ds(s * _RPS, _RPS)])
        base = s * _PAIR_R + c * _R0
        plsc.subcore_barrier()

        for hh in range(3):
            off = base + hh * _HB
            pltpu.sync_copy(srcp_hbm.at[pl.ds(off, _HB)], src_v)
            pltpu.sync_copy(dstp_hbm.at[pl.ds(off, _HB)], dst_v)
            trip = _HB // 4 if hh < 2 else (_R0 - 2 * _HB) // 4

            bufs = ((rows_a, sem_ga, sem_sa), (rows_b, sem_gb, sem_sb),
                    (rows_c, sem_gc, sem_sc), (rows_d, sem_gd, sem_sd))

            for k, (rv, sg, _) in enumerate(bufs):
                pltpu.async_copy(h_hbm.at[src_v.at[k]], rv, sg)

            def body(g, carry):
                j = g * 4
                scats = []
                for k, (rv, sg, ss) in enumerate(bufs):
                    pltpu.make_async_copy(h_hbm.at[src_v.at[j + k]], rv,
                                          sg).wait()
                    scats.append(pltpu.async_copy(
                        rv, accum.at[dst_v.at[j + k]], ss, add=True))
                for k, (rv, sg, _) in enumerate(bufs):
                    scats[k].wait()
                    jl = j + 4 + k
                    if hh < 2:
                        jl = lax.select(jl >= _HB, jl - _HB, jl)
                    pltpu.async_copy(h_hbm.at[src_v.at[jl]], rv, sg)
                return carry

            lax.fori_loop(0, trip, body, 0)
            for rv, sg, _ in bufs:
                pltpu.make_async_copy(h_hbm.at[src_v.at[0]], rv, sg).wait()
        plsc.subcore_barrier()
        pltpu.sync_copy(accum.at[pl.ds(s * _RPS, _RPS)],
                        out_hbm.at[c].at[pl.ds(s * _RPS, _RPS)])

    return agg


_agg = _make_agg()


def _mlp_body(eps_ref, h_ref, p_ref, w0_ref, b0_ref, w1_ref, b1_ref, out_ref):
    z = eps_ref[0, 0] * h_ref[...] + p_ref[0] + p_ref[1]
    t = jnp.dot(z, w0_ref[...], preferred_element_type=jnp.float32)
    t = jnp.maximum(t + b0_ref[...], 0.0)
    u = jnp.dot(t, w1_ref[...], preferred_element_type=jnp.float32)
    out_ref[...] = jnp.maximum(u + b1_ref[...], 0.0)


def _mlp(epsp, h, p, w0, b0, w1, b1):
    return pl.pallas_call(
        _mlp_body,
        grid=(_NB,),
        in_specs=[
            pl.BlockSpec((1, 1), lambda i: (0, 0), memory_space=pltpu.SMEM),
            pl.BlockSpec((_BR, _D), lambda i: (i, 0)),
            pl.BlockSpec((_NC, _BR, _D), lambda i: (0, i, 0)),
            pl.BlockSpec((_D, _D), lambda i: (0, 0)),
            pl.BlockSpec((1, _D), lambda i: (0, 0)),
            pl.BlockSpec((_D, _D), lambda i: (0, 0)),
            pl.BlockSpec((1, _D), lambda i: (0, 0)),
        ],
        out_specs=pl.BlockSpec((_BR, _D), lambda i: (i, 0)),
        out_shape=jax.ShapeDtypeStruct((_N, _D), jnp.float32),
    )(epsp, h, p, w0, b0, w1, b1)


def _pool_body(batch_ref, x_ref, h1_ref, h2_ref, wp0_ref, wp1_ref, wp2_ref,
               bsum_ref, out_ref):
    i = pl.program_id(0)
    s = jnp.dot(x_ref[...], wp0_ref[...], preferred_element_type=jnp.float32)
    s += jnp.dot(h1_ref[...], wp1_ref[...], preferred_element_type=jnp.float32)
    s += jnp.dot(h2_ref[...], wp2_ref[...], preferred_element_type=jnp.float32)
    bid = batch_ref[0, 0, :]
    onehot = (bid[:, None] == lax.broadcasted_iota(jnp.int32, (_BR, _G), 1))
    onehot = onehot.astype(jnp.float32)
    contrib = lax.dot_general(onehot, s, (((0,), (0,)), ((), ())),
                              preferred_element_type=jnp.float32)

    @pl.when(i == 0)
    def _():
        out_ref[...] = jnp.broadcast_to(bsum_ref[...], (_G, _O))

    out_ref[...] += contrib


def _pool(batch3, x, h1, h2, wp0, wp1, wp2, bsum):
    return pl.pallas_call(
        _pool_body,
        grid=(_NB,),
        in_specs=[
            pl.BlockSpec((1, 1, _BR), lambda i: (i, 0, 0)),
            pl.BlockSpec((_BR, _D), lambda i: (i, 0)),
            pl.BlockSpec((_BR, _D), lambda i: (i, 0)),
            pl.BlockSpec((_BR, _D), lambda i: (i, 0)),
            pl.BlockSpec((_D, _O), lambda i: (0, 0)),
            pl.BlockSpec((_D, _O), lambda i: (0, 0)),
            pl.BlockSpec((_D, _O), lambda i: (0, 0)),
            pl.BlockSpec((1, _O), lambda i: (0, 0)),
        ],
        out_specs=pl.BlockSpec((_G, _O), lambda i: (0, 0)),
        out_shape=jax.ShapeDtypeStruct((_G, _O), jnp.float32),
    )(batch3, x, h1, h2, wp0, wp1, wp2, bsum)


def kernel(x, edge_index, batch, params):
    src = edge_index[0]
    dst = edge_index[1]
    npad = _E_PAD - _E
    pad_src = jnp.arange(npad, dtype=jnp.int32) % _N
    pad_dst = _N + (jnp.arange(npad, dtype=jnp.int32) % (_N_ACC - _N))
    srcp = jnp.concatenate([src, pad_src]).reshape(_IDX_ROWS, _CHUNK)
    dstp = jnp.concatenate([dst, pad_dst]).reshape(_IDX_ROWS, _CHUNK)
    tail = _IDX_PAD - _IDX_ROWS
    srcp = jnp.concatenate([srcp, jnp.zeros((tail, _CHUNK), jnp.int32)])
    dstp = jnp.concatenate([dstp, jnp.full((tail, _CHUNK), _N, jnp.int32)])
    zeros = jnp.zeros((_N_ACC, _D), jnp.float32)
    batch3 = batch.reshape(_NB, 1, _BR)

    cbn = 1.0 / jnp.sqrt(1.0 + 1e-5)
    folded = []
    for l in range(2):
        g0 = params[f"mlp_g{l}"] * cbn
        w0 = params[f"W0_{l}"] * g0[None, :]
        b0 = (params[f"b0_{l}"] * g0 + params[f"mlp_b{l}"]).reshape(1, _D)
        g1 = params[f"g{l}"] * cbn
        w1 = params[f"W1_{l}"] * g1[None, :]
        b1 = (params[f"b1_{l}"] * g1 + params[f"b{l}"]).reshape(1, _D)
        epsp = (1.0 + params[f"eps{l}"]).reshape(1, 1)
        folded.append((epsp, w0, b0, w1, b1))

    h = x
    hidden = [x]
    for l in range(2):
        p = _agg(h, srcp, dstp, zeros)
        epsp, w0, b0, w1, b1 = folded[l]
        h = _mlp(epsp, h, p, w0, b0, w1, b1)
        hidden.append(h)

    bsum = (params["bp0"] + params["bp1"] + params["bp2"]).reshape(1, _O)
    return _pool(batch3, hidden[0], hidden[1], hidden[2],
                 params["Wp0"], params["Wp1"], params["Wp2"], bsum)


# speedup vs baseline: 3.5409x; 1.0172x over previous
import functools

import jax
import jax.numpy as jnp
from jax import lax
from jax.experimental import pallas as pl
from jax.experimental.pallas import tpu as pltpu
from jax.experimental.pallas import tpu_sc as plsc

_N = 10000
_E = 320000
_D = 128
_O = 64
_G = 128

_NC = 2
_NS = 16
_NW = _NC * _NS
_CHUNK = 64
_PAIR_R = 320
_R0 = 168
_R1 = _PAIR_R - _R0
_IDX_ROWS = _NS * _PAIR_R
_HB = 64
_IDX_PAD = 15 * _PAIR_R + _R0 + 3 * _HB
_E_PAD = _IDX_ROWS * _CHUNK
_N_ACC = 10112
_RPS = _N_ACC // _NS

_BR = 1000
_NB = _N // _BR


def _make_agg():
    mesh = plsc.VectorSubcoreMesh(
        core_axis_name="c", subcore_axis_name="s",
        num_cores=_NC, num_subcores=_NS)

    @functools.partial(
        pl.kernel,
        out_type=jax.ShapeDtypeStruct((_NC, _N_ACC, _D), jnp.float32),
        mesh=mesh,
        scratch_types=[
            pltpu.VMEM_SHARED((_N_ACC, _D), jnp.float32),
            pltpu.VMEM((_HB, _CHUNK), jnp.int32),
            pltpu.VMEM((_HB, _CHUNK), jnp.int32),
            pltpu.VMEM((_CHUNK, _D), jnp.float32),
            pltpu.VMEM((_CHUNK, _D), jnp.float32),
            pltpu.VMEM((_CHUNK, _D), jnp.float32),
            pltpu.VMEM((_CHUNK, _D), jnp.float32),
            pltpu.SemaphoreType.DMA,
            pltpu.SemaphoreType.DMA,
            pltpu.SemaphoreType.DMA,
            pltpu.SemaphoreType.DMA,
            pltpu.SemaphoreType.DMA,
            pltpu.SemaphoreType.DMA,
            pltpu.SemaphoreType.DMA,
            pltpu.SemaphoreType.DMA,
        ],
    )
    def agg(h_hbm, srcp_hbm, dstp_hbm, zeros_hbm, out_hbm,
            accum, src_v, dst_v, rows_a, rows_b, rows_c, rows_d,
            sem_ga, sem_gb, sem_gc, sem_gd,
            sem_sa, sem_sb, sem_sc, sem_sd):
        c = lax.axis_index("c")
        s = lax.axis_index("s")
        pltpu.sync_copy(zeros_hbm.at[pl.ds(s * _RPS, _RPS)],
                        accum.at[pl.ds(s * _RPS, _RPS)])
        base = s * _PAIR_R + c * _R0
        plsc.subcore_barrier()

        for hh in range(3):
            off = base + hh * _HB
            pltpu.sync_copy(srcp_hbm.at[pl.ds(off, _HB)], src_v)
            pltpu.sync_copy(dstp_hbm.at[pl.ds(off, _HB)], dst_v)
            if hh < 2:
                trip = _HB // 4
            else:
                trip = lax.select(c == 0, (_R0 - 2 * _HB) // 4,
                                  (_R1 - 2 * _HB) // 4)

            bufs = ((rows_a, sem_ga, sem_sa), (rows_b, sem_gb, sem_sb),
                    (rows_c, sem_gc, sem_sc), (rows_d, sem_gd, sem_sd))

            for k, (rv, sg, _) in enumerate(bufs):
                pltpu.async_copy(h_hbm.at[src_v.at[k]], rv, sg)

            def body(g, carry):
                j = g * 4
                scats = []
                for k, (rv, sg, ss) in enumerate(bufs):
                    pltpu.make_async_copy(h_hbm.at[src_v.at[j + k]], rv,
                                          sg).wait()
                    scats.append(pltpu.async_copy(
                        rv, accum.at[dst_v.at[j + k]], ss, add=True))
                for k, (rv, sg, _) in enumerate(bufs):
                    scats[k].wait()
                    jl = j + 4 + k
                    if hh < 2:
                        jl = lax.select(jl >= _HB, jl - _HB, jl)
                    pltpu.async_copy(h_hbm.at[src_v.at[jl]], rv, sg)
                return carry

            lax.fori_loop(0, trip, body, 0)
            for rv, sg, _ in bufs:
                pltpu.make_async_copy(h_hbm.at[src_v.at[0]], rv, sg).wait()
        plsc.subcore_barrier()
        pltpu.sync_copy(accum.at[pl.ds(s * _RPS, _RPS)],
                        out_hbm.at[c].at[pl.ds(s * _RPS, _RPS)])

    return agg


_agg = _make_agg()


def _mlp_body(eps_ref, h_ref, p_ref, w0_ref, b0_ref, w1_ref, b1_ref, out_ref):
    z = eps_ref[0, 0] * h_ref[...] + p_ref[0] + p_ref[1]
    t = jnp.dot(z, w0_ref[...], preferred_element_type=jnp.float32)
    t = jnp.maximum(t + b0_ref[...], 0.0)
    u = jnp.dot(t, w1_ref[...], preferred_element_type=jnp.float32)
    out_ref[...] = jnp.maximum(u + b1_ref[...], 0.0)


def _mlp(epsp, h, p, w0, b0, w1, b1):
    return pl.pallas_call(
        _mlp_body,
        grid=(_NB,),
        in_specs=[
            pl.BlockSpec((1, 1), lambda i: (0, 0), memory_space=pltpu.SMEM),
            pl.BlockSpec((_BR, _D), lambda i: (i, 0)),
            pl.BlockSpec((_NC, _BR, _D), lambda i: (0, i, 0)),
            pl.BlockSpec((_D, _D), lambda i: (0, 0)),
            pl.BlockSpec((1, _D), lambda i: (0, 0)),
            pl.BlockSpec((_D, _D), lambda i: (0, 0)),
            pl.BlockSpec((1, _D), lambda i: (0, 0)),
        ],
        out_specs=pl.BlockSpec((_BR, _D), lambda i: (i, 0)),
        out_shape=jax.ShapeDtypeStruct((_N, _D), jnp.float32),
    )(epsp, h, p, w0, b0, w1, b1)


def _pool_body(batch_ref, x_ref, h1_ref, h2_ref, wp0_ref, wp1_ref, wp2_ref,
               bsum_ref, out_ref):
    i = pl.program_id(0)
    s = jnp.dot(x_ref[...], wp0_ref[...], preferred_element_type=jnp.float32)
    s += jnp.dot(h1_ref[...], wp1_ref[...], preferred_element_type=jnp.float32)
    s += jnp.dot(h2_ref[...], wp2_ref[...], preferred_element_type=jnp.float32)
    bid = batch_ref[0, 0, :]
    onehot = (bid[:, None] == lax.broadcasted_iota(jnp.int32, (_BR, _G), 1))
    onehot = onehot.astype(jnp.float32)
    contrib = lax.dot_general(onehot, s, (((0,), (0,)), ((), ())),
                              preferred_element_type=jnp.float32)

    @pl.when(i == 0)
    def _():
        out_ref[...] = jnp.broadcast_to(bsum_ref[...], (_G, _O))

    out_ref[...] += contrib


def _pool(batch3, x, h1, h2, wp0, wp1, wp2, bsum):
    return pl.pallas_call(
        _pool_body,
        grid=(_NB,),
        in_specs=[
            pl.BlockSpec((1, 1, _BR), lambda i: (i, 0, 0)),
            pl.BlockSpec((_BR, _D), lambda i: (i, 0)),
            pl.BlockSpec((_BR, _D), lambda i: (i, 0)),
            pl.BlockSpec((_BR, _D), lambda i: (i, 0)),
            pl.BlockSpec((_D, _O), lambda i: (0, 0)),
            pl.BlockSpec((_D, _O), lambda i: (0, 0)),
            pl.BlockSpec((_D, _O), lambda i: (0, 0)),
            pl.BlockSpec((1, _O), lambda i: (0, 0)),
        ],
        out_specs=pl.BlockSpec((_G, _O), lambda i: (0, 0)),
        out_shape=jax.ShapeDtypeStruct((_G, _O), jnp.float32),
    )(batch3, x, h1, h2, wp0, wp1, wp2, bsum)


def kernel(x, edge_index, batch, params):
    src = edge_index[0]
    dst = edge_index[1]
    npad = _E_PAD - _E
    pad_src = jnp.arange(npad, dtype=jnp.int32) % _N
    pad_dst = _N + (jnp.arange(npad, dtype=jnp.int32) % (_N_ACC - _N))
    srcp = jnp.concatenate([src, pad_src]).reshape(_IDX_ROWS, _CHUNK)
    dstp = jnp.concatenate([dst, pad_dst]).reshape(_IDX_ROWS, _CHUNK)
    tail = _IDX_PAD - _IDX_ROWS
    srcp = jnp.concatenate([srcp, jnp.zeros((tail, _CHUNK), jnp.int32)])
    dstp = jnp.concatenate([dstp, jnp.full((tail, _CHUNK), _N, jnp.int32)])
    zeros = jnp.zeros((_N_ACC, _D), jnp.float32)
    batch3 = batch.reshape(_NB, 1, _BR)

    cbn = 1.0 / jnp.sqrt(1.0 + 1e-5)
    folded = []
    for l in range(2):
        g0 = params[f"mlp_g{l}"] * cbn
        w0 = params[f"W0_{l}"] * g0[None, :]
        b0 = (params[f"b0_{l}"] * g0 + params[f"mlp_b{l}"]).reshape(1, _D)
        g1 = params[f"g{l}"] * cbn
        w1 = params[f"W1_{l}"] * g1[None, :]
        b1 = (params[f"b1_{l}"] * g1 + params[f"b{l}"]).reshape(1, _D)
        epsp = (1.0 + params[f"eps{l}"]).reshape(1, 1)
        folded.append((epsp, w0, b0, w1, b1))

    h = x
    hidden = [x]
    for l in range(2):
        p = _agg(h, srcp, dstp, zeros)
        epsp, w0, b0, w1, b1 = folded[l]
        h = _mlp(epsp, h, p, w0, b0, w1, b1)
        hidden.append(h)

    bsum = (params["bp0"] + params["bp1"] + params["bp2"]).reshape(1, _O)
    return _pool(batch3, hidden[0], hidden[1], hidden[2],
                 params["Wp0"], params["Wp1"], params["Wp2"], bsum)
